# Initial kernel scaffold; baseline (speedup 1.0000x reference)
import jax
import jax.numpy as jnp
from jax.experimental import pallas as pl


def kernel(x, edge_index, edge_weight, W1, b1, Wc, bc, W2, b2):
    raise NotImplementedError("write your pallas kernel here")


# speedup vs baseline: 27.2080x; 27.2080x over previous
import functools

import jax
import jax.numpy as jnp
from jax import lax
from jax.experimental import pallas as pl
from jax.experimental.pallas import tpu as pltpu
from jax.experimental.pallas import tpu_sc as plsc

N = 100000
F_IN = 128
H = 16
C = 2

NC = 2
NS = 16
NW = NC * NS
LANES = 128
K = 8
NP = 114688
S_NP = NP // NS
NH = 50000
S_NH = NH // NS

_mesh = plsc.VectorSubcoreMesh(
    core_axis_name="c", subcore_axis_name="s", num_cores=NC, num_subcores=NS)
_mesh1 = plsc.VectorSubcoreMesh(
    core_axis_name="c", subcore_axis_name="s", num_cores=1, num_subcores=NS)


def _make_deg_kernel(steps):
    @functools.partial(
        pl.kernel,
        out_type=(jax.ShapeDtypeStruct((NP,), jnp.float32),
                  jax.ShapeDtypeStruct((NP,), jnp.float32)),
        mesh=_mesh,
        scratch_types=[
            pltpu.VMEM((K, LANES), jnp.int32),
            pltpu.VMEM((K, LANES), jnp.float32),
            pltpu.VMEM((S_NP,), jnp.float32),
            pltpu.VMEM_SHARED((NP,), jnp.float32),
        ],
    )
    def deg_kernel(col_hbm, ew_hbm, ones_hbm, zeros_hbm, out0_hbm, out1_hbm,
                   cidx_v, ew_v, stripe_v, acc_sh):
        cid = lax.axis_index("c")
        sid = lax.axis_index("s")
        wid = sid * NC + cid
        @pl.when(cid == 0)
        def _():
            pltpu.sync_copy(ones_hbm.at[pl.ds(sid * S_NP, S_NP)], stripe_v)

        @pl.when(cid == 1)
        def _():
            pltpu.sync_copy(zeros_hbm.at[pl.ds(sid * S_NP, S_NP)], stripe_v)

        pltpu.sync_copy(stripe_v, acc_sh.at[pl.ds(sid * S_NP, S_NP)])
        plsc.subcore_barrier()

        def body(step, carry):
            base = step * K
            pltpu.sync_copy(col_hbm.at[wid, pl.ds(base, K)], cidx_v)
            pltpu.sync_copy(ew_hbm.at[wid, pl.ds(base, K)], ew_v)
            for j in range(K):
                pltpu.sync_copy(ew_v.at[j], acc_sh.at[cidx_v.at[j]], add=True)
            return carry

        lax.fori_loop(0, steps, body, 0)
        plsc.subcore_barrier()
        pltpu.sync_copy(acc_sh.at[pl.ds(sid * S_NP, S_NP)], stripe_v)

        @pl.when(cid == 0)
        def _():
            pltpu.sync_copy(stripe_v, out0_hbm.at[pl.ds(sid * S_NP, S_NP)])

        @pl.when(cid == 1)
        def _():
            pltpu.sync_copy(stripe_v, out1_hbm.at[pl.ds(sid * S_NP, S_NP)])

    return deg_kernel


def _make_agg_kernel(steps):
    @functools.partial(
        pl.kernel,
        out_type=(jax.ShapeDtypeStruct((NH, H), jnp.float32),
                  jax.ShapeDtypeStruct((NH, H), jnp.float32)),
        mesh=_mesh,
        scratch_types=[
            pltpu.VMEM((K, LANES), jnp.int32),
            pltpu.VMEM((K, LANES), jnp.int32),
            pltpu.VMEM((K, LANES), jnp.float32),
            pltpu.VMEM((K, LANES, H), jnp.float32),
            pltpu.VMEM((S_NH, H), jnp.float32),
            pltpu.VMEM_SHARED((NH, H), jnp.float32),
            pltpu.SemaphoreType.DMA,
        ],
        compiler_params=pltpu.CompilerParams(use_tc_tiling_on_sc=False),
    )
    def agg_kernel(row_hbm, col_hbm, ew_hbm, g_hbm, out0_hbm, out1_hbm,
                   ridx_v, cidx_v, ew_v, rows_v, stripe_v, acc_sh, sem):
        cid = lax.axis_index("c")
        sid = lax.axis_index("s")
        nbase = cid * NH
        zv = jnp.zeros((16,), jnp.float32)

        def zero_body(i, c):
            stripe_v[i, :] = zv
            return c

        lax.fori_loop(0, S_NH, zero_body, 0)
        pltpu.sync_copy(stripe_v, acc_sh.at[pl.ds(sid * S_NH, S_NH)])
        plsc.subcore_barrier()

        def body(step, carry):
            base = step * K
            pltpu.sync_copy(row_hbm.at[sid, pl.ds(base, K)], ridx_v)
            pltpu.sync_copy(col_hbm.at[sid, pl.ds(base, K)], cidx_v)
            pltpu.sync_copy(ew_hbm.at[sid, pl.ds(base, K)], ew_v)
            descs = [pltpu.async_copy(g_hbm.at[ridx_v.at[j]], rows_v.at[j], sem)
                     for j in range(K)]
            for j in range(K):
                descs[j].wait()

                def mul_body(i, c):
                    base_i = i * 16
                    col16 = cidx_v[j, pl.ds(base_i, 16)]
                    local = col16 - nbase
                    m = (local >= 0) & (local < NH)
                    cidx_v[j, pl.ds(base_i, 16)] = jnp.where(m, local, 0)
                    ew16 = jnp.where(m, ew_v[j, pl.ds(base_i, 16)], 0.0)
                    for l in range(16):
                        rows_v[j, base_i + l, :] = rows_v[j, base_i + l, :] * ew16[l]
                    return c

                lax.fori_loop(0, LANES // 16, mul_body, 0)
                pltpu.sync_copy(rows_v.at[j], acc_sh.at[cidx_v.at[j]], add=True)
            return carry

        lax.fori_loop(0, steps, body, 0)
        plsc.subcore_barrier()
        pltpu.sync_copy(acc_sh.at[pl.ds(sid * S_NH, S_NH)], stripe_v)

        @pl.when(cid == 0)
        def _():
            pltpu.sync_copy(stripe_v, out0_hbm.at[pl.ds(sid * S_NH, S_NH)])

        @pl.when(cid == 1)
        def _():
            pltpu.sync_copy(stripe_v, out1_hbm.at[pl.ds(sid * S_NH, S_NH)])

    return agg_kernel


_BN = 2000
_HB = N // (2 * _BN)


def _tc1_body(x_ref, w1_ref, b1_ref, dp_ref, g_ref):
    deg = dp_ref[:, 0:1] + dp_ref[:, 1:2]
    dis = lax.rsqrt(deg)
    h = jnp.dot(x_ref[...], w1_ref[...],
                preferred_element_type=jnp.float32,
                precision=lax.Precision.HIGHEST)
    h = jnp.maximum(h + b1_ref[...][None, :], 0.0)
    g_ref[...] = h * dis


def _tc2_body(b0_ref, b1_ref, g_ref, dp_ref, wc_ref, bc_ref, w2_ref, b2_ref,
              out_ref):
    i = pl.program_id(0)
    deg = dp_ref[:, 0:1] + dp_ref[:, 1:2]
    dis = lax.rsqrt(deg)
    bp = jnp.where(i < N // (2 * _BN), b0_ref[...], b1_ref[...])
    b_tot = bp + g_ref[...]
    agg = b_tot * dis
    h2 = jnp.dot(agg, wc_ref[...],
                 preferred_element_type=jnp.float32,
                 precision=lax.Precision.HIGHEST)
    h2 = jnp.maximum(h2 + bc_ref[...][None, :], 0.0)
    out = jnp.dot(h2, w2_ref[...],
                  preferred_element_type=jnp.float32,
                  precision=lax.Precision.HIGHEST)
    out_ref[...] = out + b2_ref[...][None, :]


def kernel(x, edge_index, edge_weight, W1, b1, Wc, bc, W2, b2):
    E = edge_index.shape[1]
    chunk = NW * LANES * K
    steps = -(-E // chunk)
    EP = steps * chunk
    T = EP // (NW * LANES)

    pad = EP - E
    row = jnp.concatenate([edge_index[0], jnp.zeros((pad,), jnp.int32)])
    col = jnp.concatenate([edge_index[1], jnp.zeros((pad,), jnp.int32)])
    ew = jnp.concatenate([edge_weight, jnp.zeros((pad,), jnp.float32)])
    row = row.reshape(NW, T, LANES)
    col = col.reshape(NW, T, LANES)
    ew = ew.reshape(NW, T, LANES)

    ones_np = jnp.ones((NP,), jnp.float32)
    zeros_np = jnp.zeros((NP,), jnp.float32)
    d0, d1 = _make_deg_kernel(T // K)(col, ew, ones_np, zeros_np)
    dp_t = jnp.stack([d0[:N], d1[:N]], axis=1)

    nblocks = N // _BN
    g = pl.pallas_call(
        _tc1_body,
        grid=(nblocks,),
        in_specs=[
            pl.BlockSpec((_BN, F_IN), lambda i: (i, 0)),
            pl.BlockSpec((F_IN, H), lambda i: (0, 0)),
            pl.BlockSpec((H,), lambda i: (0,)),
            pl.BlockSpec((_BN, NC), lambda i: (i, 0)),
        ],
        out_specs=pl.BlockSpec((_BN, H), lambda i: (i, 0)),
        out_shape=jax.ShapeDtypeStruct((N, H), jnp.float32),
    )(x, W1, b1, dp_t)

    row1 = row.reshape(NS, 2 * T, LANES)
    col1 = col.reshape(NS, 2 * T, LANES)
    ew1 = ew.reshape(NS, 2 * T, LANES)
    bp0, bp1 = _make_agg_kernel(2 * T // K)(row1, col1, ew1, g)

    out = pl.pallas_call(
        _tc2_body,
        grid=(nblocks,),
        in_specs=[
            pl.BlockSpec((_BN, H), lambda i: (jnp.where(i < _HB, i, 0), 0)),
            pl.BlockSpec((_BN, H), lambda i: (jnp.where(i >= _HB, i - _HB, 0), 0)),
            pl.BlockSpec((_BN, H), lambda i: (i, 0)),
            pl.BlockSpec((_BN, NC), lambda i: (i, 0)),
            pl.BlockSpec((H, H), lambda i: (0, 0)),
            pl.BlockSpec((H,), lambda i: (0,)),
            pl.BlockSpec((H, C), lambda i: (0, 0)),
            pl.BlockSpec((C,), lambda i: (0,)),
        ],
        out_specs=pl.BlockSpec((_BN, C), lambda i: (i, 0)),
        out_shape=jax.ShapeDtypeStruct((N, C), jnp.float32),
    )(bp0, bp1, g, dp_t, Wc, bc, W2, b2)
    return out


# speedup vs baseline: 27.3339x; 1.0046x over previous
import functools

import jax
import jax.numpy as jnp
from jax import lax
from jax.experimental import pallas as pl
from jax.experimental.pallas import tpu as pltpu
from jax.experimental.pallas import tpu_sc as plsc

N = 100000
F_IN = 128
H = 16
C = 2

NC = 2
NS = 16
NW = NC * NS
LANES = 128
K = 8
NP = 114688
S_NP = NP // NS
NH = 50000
S_NH = NH // NS
SCH = 625

_mesh = plsc.VectorSubcoreMesh(
    core_axis_name="c", subcore_axis_name="s", num_cores=NC, num_subcores=NS)


def _make_deg_kernel(steps):
    @functools.partial(
        pl.kernel,
        out_type=(jax.ShapeDtypeStruct((NP,), jnp.float32),
                  jax.ShapeDtypeStruct((NP,), jnp.float32)),
        mesh=_mesh,
        scratch_types=[
            pltpu.VMEM((K, LANES), jnp.int32),
            pltpu.VMEM((K, LANES), jnp.float32),
            pltpu.VMEM((S_NP,), jnp.float32),
            pltpu.VMEM_SHARED((NP,), jnp.float32),
        ],
    )
    def deg_kernel(col_hbm, ew_hbm, ones_hbm, zeros_hbm, out0_hbm, out1_hbm,
                   cidx_v, ew_v, stripe_v, acc_sh):
        cid = lax.axis_index("c")
        sid = lax.axis_index("s")
        wid = sid * NC + cid
        @pl.when(cid == 0)
        def _():
            pltpu.sync_copy(ones_hbm.at[pl.ds(sid * S_NP, S_NP)], stripe_v)

        @pl.when(cid == 1)
        def _():
            pltpu.sync_copy(zeros_hbm.at[pl.ds(sid * S_NP, S_NP)], stripe_v)

        pltpu.sync_copy(stripe_v, acc_sh.at[pl.ds(sid * S_NP, S_NP)])
        plsc.subcore_barrier()

        def body(step, carry):
            base = step * K
            pltpu.sync_copy(col_hbm.at[wid, pl.ds(base, K)], cidx_v)
            pltpu.sync_copy(ew_hbm.at[wid, pl.ds(base, K)], ew_v)
            for j in range(K):
                pltpu.sync_copy(ew_v.at[j], acc_sh.at[cidx_v.at[j]], add=True)
            return carry

        lax.fori_loop(0, steps, body, 0)
        plsc.subcore_barrier()
        pltpu.sync_copy(acc_sh.at[pl.ds(sid * S_NP, S_NP)], stripe_v)

        @pl.when(cid == 0)
        def _():
            pltpu.sync_copy(stripe_v, out0_hbm.at[pl.ds(sid * S_NP, S_NP)])

        @pl.when(cid == 1)
        def _():
            pltpu.sync_copy(stripe_v, out1_hbm.at[pl.ds(sid * S_NP, S_NP)])

    return deg_kernel


def _make_agg_kernel(steps2):
    @functools.partial(
        pl.kernel,
        out_type=(jax.ShapeDtypeStruct((NH, H), jnp.float32),
                  jax.ShapeDtypeStruct((NH, H), jnp.float32)),
        mesh=_mesh,
        scratch_types=[
            pltpu.VMEM((2, K, LANES), jnp.int32),
            pltpu.VMEM((2, K, LANES), jnp.int32),
            pltpu.VMEM((2, K, LANES), jnp.float32),
            pltpu.VMEM((2, K, LANES, H), jnp.float32),
            pltpu.VMEM((SCH, H), jnp.float32),
            pltpu.VMEM_SHARED((NH, H), jnp.float32),
            pltpu.SemaphoreType.DMA,
            pltpu.SemaphoreType.DMA,
            pltpu.SemaphoreType.DMA,
            pltpu.SemaphoreType.DMA,
            pltpu.SemaphoreType.DMA,
        ],
        compiler_params=pltpu.CompilerParams(use_tc_tiling_on_sc=False),
    )
    def agg_kernel(row_hbm, col_hbm, ew_hbm, g_hbm, out0_hbm, out1_hbm,
                   ridx_v, cidx_v, ew_v, rows_v, chunk_v, acc_sh,
                   sem_g, sem_s0, sem_s1, sem_i0, sem_i1):
        cid = lax.axis_index("c")
        sid = lax.axis_index("s")
        nbase = cid * NH
        sem_s = (sem_s0, sem_s1)
        sem_i = (sem_i0, sem_i1)
        steps = 2 * steps2

        zv = jnp.zeros((16,), jnp.float32)

        def zero_body(i, c):
            chunk_v[i, :] = zv
            return c

        lax.fori_loop(0, SCH, zero_body, 0)
        for q in range(S_NH // SCH):
            pltpu.sync_copy(chunk_v, acc_sh.at[pl.ds(sid * S_NH + q * SCH, SCH)])
        plsc.subcore_barrier()

        def idx_copies(step, b):
            base = step * K
            return [(row_hbm.at[sid, pl.ds(base, K)], ridx_v.at[b]),
                    (col_hbm.at[sid, pl.ds(base, K)], cidx_v.at[b]),
                    (ew_hbm.at[sid, pl.ds(base, K)], ew_v.at[b])]

        def issue_idx(step, b):
            for src, dst in idx_copies(step, b):
                pltpu.async_copy(src, dst, sem_i[b])

        def drain_idx(step, b):
            for src, dst in idx_copies(step, b):
                pltpu.make_async_copy(src, dst, sem_i[b]).wait()

        def drain_scatters(b):
            for j in range(K):
                pltpu.make_async_copy(rows_v.at[b, j],
                                      acc_sh.at[cidx_v.at[b, j]],
                                      sem_s[b]).wait()

        issue_idx(0, 0)

        def do_step(step, outer, b):
            drain_idx(step, b)
            descs = [pltpu.async_copy(g_hbm.at[ridx_v.at[b, j]],
                                      rows_v.at[b, j], sem_g)
                     for j in range(K)]
            if b == 0:
                @pl.when(outer > 0)
                def _():
                    drain_scatters(1)
            else:
                drain_scatters(0)

            @pl.when(step + 1 < steps)
            def _():
                issue_idx(step + 1, 1 - b)

            for j in range(K):
                descs[j].wait()

                def mul_body(i, c):
                    base_i = i * 16
                    col16 = cidx_v[b, j, pl.ds(base_i, 16)]
                    local = col16 - nbase
                    m = (local >= 0) & (local < NH)
                    cidx_v[b, j, pl.ds(base_i, 16)] = jnp.where(m, local, 0)
                    ew16 = jnp.where(m, ew_v[b, j, pl.ds(base_i, 16)], 0.0)
                    for l in range(16):
                        rows_v[b, j, base_i + l, :] = (
                            rows_v[b, j, base_i + l, :] * ew16[l])
                    return c

                lax.fori_loop(0, LANES // 16, mul_body, 0)
                pltpu.async_copy(rows_v.at[b, j], acc_sh.at[cidx_v.at[b, j]],
                                 sem_s[b], add=True)

        def body(outer, carry):
            do_step(2 * outer, outer, 0)
            do_step(2 * outer + 1, outer, 1)
            return carry

        lax.fori_loop(0, steps2, body, 0)
        drain_scatters(1)
        plsc.subcore_barrier()
        for q in range(S_NH // SCH):
            pltpu.sync_copy(acc_sh.at[pl.ds(sid * S_NH + q * SCH, SCH)],
                            chunk_v)

            @pl.when(cid == 0)
            def _():
                pltpu.sync_copy(chunk_v,
                                out0_hbm.at[pl.ds(sid * S_NH + q * SCH, SCH)])

            @pl.when(cid == 1)
            def _():
                pltpu.sync_copy(chunk_v,
                                out1_hbm.at[pl.ds(sid * S_NH + q * SCH, SCH)])

    return agg_kernel


_BN = 2000
_HB = N // (2 * _BN)


def _tc1_body(x_ref, w1_ref, b1_ref, dp_ref, g_ref):
    deg = dp_ref[:, 0:1] + dp_ref[:, 1:2]
    dis = lax.rsqrt(deg)
    h = jnp.dot(x_ref[...], w1_ref[...],
                preferred_element_type=jnp.float32,
                precision=lax.Precision.HIGHEST)
    h = jnp.maximum(h + b1_ref[...][None, :], 0.0)
    g_ref[...] = h * dis


def _tc2_body(b0_ref, b1_ref, g_ref, dp_ref, wc_ref, bc_ref, w2_ref, b2_ref,
              out_ref):
    i = pl.program_id(0)
    deg = dp_ref[:, 0:1] + dp_ref[:, 1:2]
    dis = lax.rsqrt(deg)
    bp = jnp.where(i < _HB, b0_ref[...], b1_ref[...])
    b_tot = bp + g_ref[...]
    agg = b_tot * dis
    h2 = jnp.dot(agg, wc_ref[...],
                 preferred_element_type=jnp.float32,
                 precision=lax.Precision.HIGHEST)
    h2 = jnp.maximum(h2 + bc_ref[...][None, :], 0.0)
    out = jnp.dot(h2, w2_ref[...],
                  preferred_element_type=jnp.float32,
                  precision=lax.Precision.HIGHEST)
    out_ref[...] = out + b2_ref[...][None, :]


def kernel(x, edge_index, edge_weight, W1, b1, Wc, bc, W2, b2):
    E = edge_index.shape[1]
    chunk = NW * LANES * K
    steps = -(-E // chunk)
    EP = steps * chunk
    T = EP // (NW * LANES)

    pad = EP - E
    row = jnp.concatenate([edge_index[0], jnp.zeros((pad,), jnp.int32)])
    col = jnp.concatenate([edge_index[1], jnp.zeros((pad,), jnp.int32)])
    ew = jnp.concatenate([edge_weight, jnp.zeros((pad,), jnp.float32)])
    row = row.reshape(NW, T, LANES)
    col = col.reshape(NW, T, LANES)
    ew = ew.reshape(NW, T, LANES)

    ones_np = jnp.ones((NP,), jnp.float32)
    zeros_np = jnp.zeros((NP,), jnp.float32)
    d0, d1 = _make_deg_kernel(T // K)(col, ew, ones_np, zeros_np)
    dp_t = jnp.stack([d0[:N], d1[:N]], axis=1)

    nblocks = N // _BN
    g = pl.pallas_call(
        _tc1_body,
        grid=(nblocks,),
        in_specs=[
            pl.BlockSpec((_BN, F_IN), lambda i: (i, 0)),
            pl.BlockSpec((F_IN, H), lambda i: (0, 0)),
            pl.BlockSpec((H,), lambda i: (0,)),
            pl.BlockSpec((_BN, NC), lambda i: (i, 0)),
        ],
        out_specs=pl.BlockSpec((_BN, H), lambda i: (i, 0)),
        out_shape=jax.ShapeDtypeStruct((N, H), jnp.float32),
    )(x, W1, b1, dp_t)

    row1 = row.reshape(NS, 2 * T, LANES)
    col1 = col.reshape(NS, 2 * T, LANES)
    ew1 = ew.reshape(NS, 2 * T, LANES)
    bp0, bp1 = _make_agg_kernel(T // K)(row1, col1, ew1, g)

    out = pl.pallas_call(
        _tc2_body,
        grid=(nblocks,),
        in_specs=[
            pl.BlockSpec((_BN, H), lambda i: (jnp.where(i < _HB, i, 0), 0)),
            pl.BlockSpec((_BN, H), lambda i: (jnp.where(i >= _HB, i - _HB, 0), 0)),
            pl.BlockSpec((_BN, H), lambda i: (i, 0)),
            pl.BlockSpec((_BN, NC), lambda i: (i, 0)),
            pl.BlockSpec((H, H), lambda i: (0, 0)),
            pl.BlockSpec((H,), lambda i: (0,)),
            pl.BlockSpec((H, C), lambda i: (0, 0)),
            pl.BlockSpec((C,), lambda i: (0,)),
        ],
        out_specs=pl.BlockSpec((_BN, C), lambda i: (i, 0)),
        out_shape=jax.ShapeDtypeStruct((N, C), jnp.float32),
    )(bp0, bp1, g, dp_t, Wc, bc, W2, b2)
    return out


# speedup vs baseline: 28.6709x; 1.0489x over previous
import functools

import jax
import jax.numpy as jnp
from jax import lax
from jax.experimental import pallas as pl
from jax.experimental.pallas import tpu as pltpu
from jax.experimental.pallas import tpu_sc as plsc

N = 100000
F_IN = 128
H = 16
C = 2

NC = 2
NS = 16
NW = NC * NS
LANES = 128
K = 8
NP = 114688
S_NP = NP // NS
NH = 50000
S_NH = NH // NS
SCH = 625

_mesh = plsc.VectorSubcoreMesh(
    core_axis_name="c", subcore_axis_name="s", num_cores=NC, num_subcores=NS)


def _make_deg_kernel(steps2):
    @functools.partial(
        pl.kernel,
        out_type=(jax.ShapeDtypeStruct((NP,), jnp.float32),
                  jax.ShapeDtypeStruct((NP,), jnp.float32)),
        mesh=_mesh,
        scratch_types=[
            pltpu.VMEM((2, K, LANES), jnp.int32),
            pltpu.VMEM((2, K, LANES), jnp.float32),
            pltpu.VMEM((S_NP,), jnp.float32),
            pltpu.VMEM_SHARED((NP,), jnp.float32),
            pltpu.SemaphoreType.DMA,
            pltpu.SemaphoreType.DMA,
            pltpu.SemaphoreType.DMA,
            pltpu.SemaphoreType.DMA,
        ],
    )
    def deg_kernel(col_hbm, ew_hbm, ones_hbm, zeros_hbm, out0_hbm, out1_hbm,
                   cidx_v, ew_v, stripe_v, acc_sh,
                   sem_s0, sem_s1, sem_i0, sem_i1):
        cid = lax.axis_index("c")
        sid = lax.axis_index("s")
        wid = sid * NC + cid
        sem_s = (sem_s0, sem_s1)
        sem_i = (sem_i0, sem_i1)
        steps = 2 * steps2
        @pl.when(cid == 0)
        def _():
            pltpu.sync_copy(ones_hbm.at[pl.ds(sid * S_NP, S_NP)], stripe_v)

        @pl.when(cid == 1)
        def _():
            pltpu.sync_copy(zeros_hbm.at[pl.ds(sid * S_NP, S_NP)], stripe_v)

        pltpu.sync_copy(stripe_v, acc_sh.at[pl.ds(sid * S_NP, S_NP)])
        plsc.subcore_barrier()

        def idx_copies(step, b):
            base = step * K
            return [(col_hbm.at[wid, pl.ds(base, K)], cidx_v.at[b]),
                    (ew_hbm.at[wid, pl.ds(base, K)], ew_v.at[b])]

        def issue_idx(step, b):
            for src, dst in idx_copies(step, b):
                pltpu.async_copy(src, dst, sem_i[b])

        def drain_idx(step, b):
            for src, dst in idx_copies(step, b):
                pltpu.make_async_copy(src, dst, sem_i[b]).wait()

        def drain_scatters(b):
            for j in range(K):
                pltpu.make_async_copy(ew_v.at[b, j],
                                      acc_sh.at[cidx_v.at[b, j]],
                                      sem_s[b]).wait()

        issue_idx(0, 0)

        def do_step(step, outer, b):
            drain_idx(step, b)
            if b == 0:
                @pl.when(outer > 0)
                def _():
                    drain_scatters(1)
            else:
                drain_scatters(0)

            @pl.when(step + 1 < steps)
            def _():
                issue_idx(step + 1, 1 - b)

            for j in range(K):
                pltpu.async_copy(ew_v.at[b, j], acc_sh.at[cidx_v.at[b, j]],
                                 sem_s[b], add=True)

        def body(outer, carry):
            do_step(2 * outer, outer, 0)
            do_step(2 * outer + 1, outer, 1)
            return carry

        lax.fori_loop(0, steps2, body, 0)
        drain_scatters(1)
        plsc.subcore_barrier()
        pltpu.sync_copy(acc_sh.at[pl.ds(sid * S_NP, S_NP)], stripe_v)

        @pl.when(cid == 0)
        def _():
            pltpu.sync_copy(stripe_v, out0_hbm.at[pl.ds(sid * S_NP, S_NP)])

        @pl.when(cid == 1)
        def _():
            pltpu.sync_copy(stripe_v, out1_hbm.at[pl.ds(sid * S_NP, S_NP)])

    return deg_kernel


def _make_agg_kernel(steps2):
    @functools.partial(
        pl.kernel,
        out_type=(jax.ShapeDtypeStruct((NH, H), jnp.float32),
                  jax.ShapeDtypeStruct((NH, H), jnp.float32)),
        mesh=_mesh,
        scratch_types=[
            pltpu.VMEM((2, K, LANES), jnp.int32),
            pltpu.VMEM((2, K, LANES), jnp.int32),
            pltpu.VMEM((2, K, LANES), jnp.float32),
            pltpu.VMEM((2, K, LANES, H), jnp.float32),
            pltpu.VMEM((SCH, H), jnp.float32),
            pltpu.VMEM_SHARED((NH, H), jnp.float32),
            pltpu.SemaphoreType.DMA,
            pltpu.SemaphoreType.DMA,
            pltpu.SemaphoreType.DMA,
            pltpu.SemaphoreType.DMA,
            pltpu.SemaphoreType.DMA,
        ],
        compiler_params=pltpu.CompilerParams(use_tc_tiling_on_sc=False),
    )
    def agg_kernel(row_hbm, col_hbm, ew_hbm, g_hbm, out0_hbm, out1_hbm,
                   ridx_v, cidx_v, ew_v, rows_v, chunk_v, acc_sh,
                   sem_g, sem_s0, sem_s1, sem_i0, sem_i1):
        cid = lax.axis_index("c")
        sid = lax.axis_index("s")
        nbase = cid * NH
        sem_s = (sem_s0, sem_s1)
        sem_i = (sem_i0, sem_i1)
        steps = 2 * steps2

        zv = jnp.zeros((16,), jnp.float32)

        def zero_body(i, c):
            chunk_v[i, :] = zv
            return c

        lax.fori_loop(0, SCH, zero_body, 0)
        for q in range(S_NH // SCH):
            pltpu.sync_copy(chunk_v, acc_sh.at[pl.ds(sid * S_NH + q * SCH, SCH)])
        plsc.subcore_barrier()

        def idx_copies(step, b):
            base = step * K
            return [(row_hbm.at[sid, pl.ds(base, K)], ridx_v.at[b]),
                    (col_hbm.at[sid, pl.ds(base, K)], cidx_v.at[b]),
                    (ew_hbm.at[sid, pl.ds(base, K)], ew_v.at[b])]

        def issue_idx(step, b):
            for src, dst in idx_copies(step, b):
                pltpu.async_copy(src, dst, sem_i[b])

        def drain_idx(step, b):
            for src, dst in idx_copies(step, b):
                pltpu.make_async_copy(src, dst, sem_i[b]).wait()

        def drain_scatters(b):
            for j in range(K):
                pltpu.make_async_copy(rows_v.at[b, j],
                                      acc_sh.at[cidx_v.at[b, j]],
                                      sem_s[b]).wait()

        issue_idx(0, 0)

        def do_step(step, outer, b):
            drain_idx(step, b)
            descs = [pltpu.async_copy(g_hbm.at[ridx_v.at[b, j]],
                                      rows_v.at[b, j], sem_g)
                     for j in range(K)]
            if b == 0:
                @pl.when(outer > 0)
                def _():
                    drain_scatters(1)
            else:
                drain_scatters(0)

            @pl.when(step + 1 < steps)
            def _():
                issue_idx(step + 1, 1 - b)

            for j in range(K):
                descs[j].wait()

                def mul_body(i, c):
                    base_i = i * 16
                    col16 = cidx_v[b, j, pl.ds(base_i, 16)]
                    local = col16 - nbase
                    m = (local >= 0) & (local < NH)
                    cidx_v[b, j, pl.ds(base_i, 16)] = jnp.where(m, local, 0)
                    ew16 = jnp.where(m, ew_v[b, j, pl.ds(base_i, 16)], 0.0)
                    for l in range(16):
                        rows_v[b, j, base_i + l, :] = (
                            rows_v[b, j, base_i + l, :] * ew16[l])
                    return c

                lax.fori_loop(0, LANES // 16, mul_body, 0)
                pltpu.async_copy(rows_v.at[b, j], acc_sh.at[cidx_v.at[b, j]],
                                 sem_s[b], add=True)

        def body(outer, carry):
            do_step(2 * outer, outer, 0)
            do_step(2 * outer + 1, outer, 1)
            return carry

        lax.fori_loop(0, steps2, body, 0)
        drain_scatters(1)
        plsc.subcore_barrier()
        for q in range(S_NH // SCH):
            pltpu.sync_copy(acc_sh.at[pl.ds(sid * S_NH + q * SCH, SCH)],
                            chunk_v)

            @pl.when(cid == 0)
            def _():
                pltpu.sync_copy(chunk_v,
                                out0_hbm.at[pl.ds(sid * S_NH + q * SCH, SCH)])

            @pl.when(cid == 1)
            def _():
                pltpu.sync_copy(chunk_v,
                                out1_hbm.at[pl.ds(sid * S_NH + q * SCH, SCH)])

    return agg_kernel


_BN = 2000
_HB = N // (2 * _BN)


def _tc1a_body(x_ref, w1_ref, b1_ref, h_ref):
    h = jnp.dot(x_ref[...], w1_ref[...],
                preferred_element_type=jnp.float32,
                precision=lax.Precision.HIGHEST)
    h_ref[...] = jnp.maximum(h + b1_ref[...][None, :], 0.0)


def _tc1b_body(h_ref, dp_ref, g_ref):
    deg = dp_ref[:, 0:1] + dp_ref[:, 1:2]
    dis = lax.rsqrt(deg)
    g_ref[...] = h_ref[...] * dis


def _tc2_body(b0_ref, b1_ref, g_ref, dp_ref, wc_ref, bc_ref, w2_ref, b2_ref,
              out_ref):
    i = pl.program_id(0)
    deg = dp_ref[:, 0:1] + dp_ref[:, 1:2]
    dis = lax.rsqrt(deg)
    bp = jnp.where(i < _HB, b0_ref[...], b1_ref[...])
    b_tot = bp + g_ref[...]
    agg = b_tot * dis
    h2 = jnp.dot(agg, wc_ref[...],
                 preferred_element_type=jnp.float32,
                 precision=lax.Precision.HIGHEST)
    h2 = jnp.maximum(h2 + bc_ref[...][None, :], 0.0)
    out = jnp.dot(h2, w2_ref[...],
                  preferred_element_type=jnp.float32,
                  precision=lax.Precision.HIGHEST)
    out_ref[...] = out + b2_ref[...][None, :]


def kernel(x, edge_index, edge_weight, W1, b1, Wc, bc, W2, b2):
    E = edge_index.shape[1]
    chunk = NW * LANES * K
    steps = -(-E // chunk)
    EP = steps * chunk
    T = EP // (NW * LANES)

    pad = EP - E
    row = jnp.concatenate([edge_index[0], jnp.zeros((pad,), jnp.int32)])
    col = jnp.concatenate([edge_index[1], jnp.zeros((pad,), jnp.int32)])
    ew = jnp.concatenate([edge_weight, jnp.zeros((pad,), jnp.float32)])
    row = row.reshape(NW, T, LANES)
    col = col.reshape(NW, T, LANES)
    ew = ew.reshape(NW, T, LANES)

    ones_np = jnp.ones((NP,), jnp.float32)
    zeros_np = jnp.zeros((NP,), jnp.float32)
    d0, d1 = _make_deg_kernel(T // K // 2)(col, ew, ones_np, zeros_np)
    dp_t = jnp.stack([d0[:N], d1[:N]], axis=1)

    nblocks = N // _BN
    h = pl.pallas_call(
        _tc1a_body,
        grid=(nblocks,),
        in_specs=[
            pl.BlockSpec((_BN, F_IN), lambda i: (i, 0)),
            pl.BlockSpec((F_IN, H), lambda i: (0, 0)),
            pl.BlockSpec((H,), lambda i: (0,)),
        ],
        out_specs=pl.BlockSpec((_BN, H), lambda i: (i, 0)),
        out_shape=jax.ShapeDtypeStruct((N, H), jnp.float32),
    )(x, W1, b1)

    g = pl.pallas_call(
        _tc1b_body,
        grid=(nblocks,),
        in_specs=[
            pl.BlockSpec((_BN, H), lambda i: (i, 0)),
            pl.BlockSpec((_BN, NC), lambda i: (i, 0)),
        ],
        out_specs=pl.BlockSpec((_BN, H), lambda i: (i, 0)),
        out_shape=jax.ShapeDtypeStruct((N, H), jnp.float32),
    )(h, dp_t)

    row1 = row.reshape(NS, 2 * T, LANES)
    col1 = col.reshape(NS, 2 * T, LANES)
    ew1 = ew.reshape(NS, 2 * T, LANES)
    bp0, bp1 = _make_agg_kernel(T // K)(row1, col1, ew1, g)

    out = pl.pallas_call(
        _tc2_body,
        grid=(nblocks,),
        in_specs=[
            pl.BlockSpec((_BN, H), lambda i: (jnp.where(i < _HB, i, 0), 0)),
            pl.BlockSpec((_BN, H), lambda i: (jnp.where(i >= _HB, i - _HB, 0), 0)),
            pl.BlockSpec((_BN, H), lambda i: (i, 0)),
            pl.BlockSpec((_BN, NC), lambda i: (i, 0)),
            pl.BlockSpec((H, H), lambda i: (0, 0)),
            pl.BlockSpec((H,), lambda i: (0,)),
            pl.BlockSpec((H, C), lambda i: (0, 0)),
            pl.BlockSpec((C,), lambda i: (0,)),
        ],
        out_specs=pl.BlockSpec((_BN, C), lambda i: (i, 0)),
        out_shape=jax.ShapeDtypeStruct((N, C), jnp.float32),
    )(bp0, bp1, g, dp_t, Wc, bc, W2, b2)
    return out
